# baseline (speedup 1.0000x reference)
import functools

import jax
import jax.numpy as jnp
from jax import lax
from jax.experimental import pallas as pl
from jax.experimental.pallas import tpu as pltpu
from jax.experimental.pallas import tpu_sc as plsc

B = 4096
L = 200
D = 64
LP = 208
NB = LP // 16
NG = D // 16

S = 3072

BR = 128

NW = 32
SC_ROWS = B - S
SC_ROWS_PER_W = SC_ROWS // NW if SC_ROWS else 0


def _tc_body(f_ref, x_ref, m_ref, o_ref):
    CR = 8

    for i in range(BR // CR):
        fr = f_ref[pl.ds(i * CR, CR)]
        xr = x_ref[pl.ds(i * CR, CR)]
        c = jnp.sum(fr * xr[:, None, :], axis=2)
        mx = jnp.max(c, axis=-1, keepdims=True)
        e = jnp.exp(c - mx)
        s = jnp.sum(e, axis=-1, keepdims=True)
        wm = (e / s) * m_ref[pl.ds(i * CR, CR)]
        o_ref[pl.ds(i * CR, CR)] = jnp.sum(wm[:, :, None] * fr, axis=1)


@functools.partial(jax.jit, static_argnames=("rows",))
def _tc_attn(f, x, m, rows):
    grid = (rows // BR,)
    return pl.pallas_call(
        _tc_body,
        grid=grid,
        in_specs=[
            pl.BlockSpec((BR, L, D), lambda i: (i, 0, 0)),
            pl.BlockSpec((BR, D), lambda i: (i, 0)),
            pl.BlockSpec((BR, L), lambda i: (i, 0)),
        ],
        out_specs=pl.BlockSpec((BR, D), lambda i: (i, 0)),
        out_shape=jax.ShapeDtypeStruct((rows, D), jnp.float32),
    )(f, x, m)


def _sc_body(f_hbm, x_hbm, m_hbm, out_hbm, f_v, x_sl, m_sl, o_sl, w_v,
             sem0, sem1):
    cid = lax.axis_index("c")
    sid = lax.axis_index("s")
    wid = cid * 16 + sid
    base = wid * SC_ROWS_PER_W

    lane = lax.iota(jnp.int32, 16)
    ones = jnp.full((16,), 1.0, jnp.float32)
    zeros = jnp.zeros((16,), jnp.float32)
    validf = jnp.where(lane < (L - 12 * 16), ones, zeros)
    l_idx = [jnp.minimum(lb * 16 + lane, L - 1) for lb in range(NB)]
    bufv = (jnp.zeros((16,), jnp.int32), jnp.full((16,), 1, jnp.int32))
    sems = (sem0, sem1)

    pltpu.sync_copy(x_hbm.at[pl.ds(base, SC_ROWS_PER_W)], x_sl)
    pltpu.sync_copy(m_hbm.at[pl.ds(base, SC_ROWS_PER_W)], m_sl)

    def start(buf, ci):
        pltpu.async_copy(f_hbm.at[pl.ds(base + ci, 1)], f_v.at[pl.ds(buf, 1)],
                         sems[buf])

    def wait(buf):
        pltpu.make_async_copy(f_hbm.at[pl.ds(0, 1)], f_v.at[pl.ds(buf, 1)],
                              sems[buf]).wait()

    def compute(buf, ci):
        civ = jnp.full((16,), ci, jnp.int32)

        @pl.loop(0, D // 4,
                 init_carry=tuple(jnp.zeros((16,), jnp.float32)
                                  for _ in range(NB)))
        def p1(i, cs):
            d0 = i * 4
            dv0 = jnp.full((16,), d0, jnp.int32)
            dvs = [dv0, dv0 + 1, dv0 + 2, dv0 + 3]
            xs = [plsc.load_gather(x_sl, [civ, dv]) for dv in dvs]
            out = []
            for lb in range(NB):
                g = [plsc.load_gather(f_v, [bufv[buf], l_idx[lb], dv])
                     for dv in dvs]
                t = (g[0] * xs[0] + g[1] * xs[1]) + (g[2] * xs[2] + g[3] * xs[3])
                out.append(cs[lb] + t)
            return tuple(out)

        cs = p1

        m_vec = cs[0]
        for lb in range(1, NB):
            m_vec = jnp.maximum(m_vec, cs[lb])
        mx = lax.broadcast_in_dim(jnp.max(m_vec), (16,), ())
        es = [jnp.exp(cs[lb] - mx) for lb in range(NB)]
        s_vec = es[NB - 1] * validf
        for lb in range(NB - 1):
            s_vec = s_vec + es[lb]
        s = lax.broadcast_in_dim(jnp.sum(s_vec), (16,), ())
        sinv = ones / s
        for lb in range(NB):
            mf = m_sl[ci, pl.ds(lb * 16, 16)]
            w_v[pl.ds(lb * 16, 16)] = es[lb] * mf * sinv

        @pl.loop(0, L // 4,
                 init_carry=tuple(jnp.zeros((16,), jnp.float32)
                                  for _ in range(4 * NG)))
        def p2(i, accs):
            l0 = i * 4
            lv0 = jnp.full((16,), l0, jnp.int32)
            accs = list(accs)
            for j in range(4):
                ws = plsc.load_gather(w_v, [lv0 + j]) if j else \
                    plsc.load_gather(w_v, [lv0])
                for g in range(NG):
                    accs[j * NG + g] = accs[j * NG + g] + ws * f_v[
                        buf, l0 + j, pl.ds(g * 16, 16)
                    ]
            return tuple(accs)

        accs = p2
        for g in range(NG):
            o_sl[ci, pl.ds(g * 16, 16)] = (
                accs[g] + accs[NG + g] + accs[2 * NG + g] + accs[3 * NG + g]
            )

    start(0, 0)

    @pl.loop(0, SC_ROWS_PER_W, step=2)
    def _chunk(ci):
        @pl.when(ci + 1 < SC_ROWS_PER_W)
        def _():
            start(1, ci + 1)
        wait(0)
        compute(0, ci)

        @pl.when(ci + 2 < SC_ROWS_PER_W)
        def _():
            start(0, ci + 2)

        @pl.when(ci + 1 < SC_ROWS_PER_W)
        def _():
            wait(1)
            compute(1, ci + 1)

    pltpu.sync_copy(o_sl, out_hbm.at[pl.ds(base, SC_ROWS_PER_W)])


def _sc_attn(f, x, mpad):
    mesh = plsc.VectorSubcoreMesh(
        core_axis_name="c", subcore_axis_name="s", num_cores=2, num_subcores=16
    )
    run = pl.kernel(
        _sc_body,
        out_type=jax.ShapeDtypeStruct((SC_ROWS, D), jnp.float32),
        mesh=mesh,
        scratch_types=[
            pltpu.VMEM((2, L, D), jnp.float32),
            pltpu.VMEM((SC_ROWS_PER_W, D), jnp.float32),
            pltpu.VMEM((SC_ROWS_PER_W, LP), jnp.float32),
            pltpu.VMEM((SC_ROWS_PER_W, D), jnp.float32),
            pltpu.VMEM((LP,), jnp.float32),
            pltpu.SemaphoreType.DMA,
            pltpu.SemaphoreType.DMA,
        ],
        compiler_params=pltpu.CompilerParams(needs_layout_passes=False),
    )
    return run(f, x, mpad)


@jax.jit
def _friendattn(f, x, m):
    mf = m.astype(jnp.float32)
    parts = []
    if S:
        parts.append(_tc_attn(f[:S], x[:S], mf[:S], rows=S))
    if SC_ROWS:
        mpad = jnp.pad(mf[S:], ((0, 0), (0, LP - L)))
        parts.append(_sc_attn(f[S:], x[S:], mpad))
    out = parts[0] if len(parts) == 1 else jnp.concatenate(parts, axis=0)
    return out.reshape(B, 1, D)


def kernel(friend_diff_x, self_x, friend_num_src, friend_num_src_tensor, friend_diff_src_mask):
    del friend_num_src, friend_num_src_tensor
    return _friendattn(friend_diff_x, self_x, friend_diff_src_mask)


# speedup vs baseline: 1.1207x; 1.1207x over previous
import functools

import jax
import jax.numpy as jnp
from jax import lax
from jax.experimental import pallas as pl
from jax.experimental.pallas import tpu as pltpu
from jax.experimental.pallas import tpu_sc as plsc

B = 4096
L = 200
D = 64
LP = 208
NB = LP // 16
NG = D // 16

S = 2048

BR = 128

NW = 32
SC_ROWS = B - S
SC_ROWS_PER_W = SC_ROWS // NW if SC_ROWS else 0


def _tc_body(f_ref, x_ref, m_ref, o_ref):
    CR = 8

    for i in range(BR // CR):
        fr = f_ref[pl.ds(i * CR, CR)]
        xr = x_ref[pl.ds(i * CR, CR)]
        c = jnp.sum(fr * xr[:, None, :], axis=2)
        mx = jnp.max(c, axis=-1, keepdims=True)
        e = jnp.exp(c - mx)
        s = jnp.sum(e, axis=-1, keepdims=True)
        wm = (e / s) * m_ref[pl.ds(i * CR, CR)]
        o_ref[pl.ds(i * CR, CR)] = jnp.sum(wm[:, :, None] * fr, axis=1)


@functools.partial(jax.jit, static_argnames=("rows",))
def _tc_attn(f, x, m, rows):
    grid = (rows // BR,)
    return pl.pallas_call(
        _tc_body,
        grid=grid,
        in_specs=[
            pl.BlockSpec((BR, L, D), lambda i: (i, 0, 0)),
            pl.BlockSpec((BR, D), lambda i: (i, 0)),
            pl.BlockSpec((BR, L), lambda i: (i, 0)),
        ],
        out_specs=pl.BlockSpec((BR, D), lambda i: (i, 0)),
        out_shape=jax.ShapeDtypeStruct((rows, D), jnp.float32),
    )(f, x, m)


def _sc_body(f_hbm, x_hbm, m_hbm, out_hbm, f_v, x_sl, m_sl, o_sl, w_v,
             sem0, sem1):
    cid = lax.axis_index("c")
    sid = lax.axis_index("s")
    wid = cid * 16 + sid
    base = wid * SC_ROWS_PER_W

    lane = lax.iota(jnp.int32, 16)
    ones = jnp.full((16,), 1.0, jnp.float32)
    zeros = jnp.zeros((16,), jnp.float32)
    validf = jnp.where(lane < (L - 12 * 16), ones, zeros)
    l_idx = [jnp.minimum(lb * 16 + lane, L - 1) for lb in range(NB)]
    bufv = (jnp.zeros((16,), jnp.int32), jnp.full((16,), 1, jnp.int32))
    sems = (sem0, sem1)

    pltpu.sync_copy(x_hbm.at[pl.ds(base, SC_ROWS_PER_W)], x_sl)
    pltpu.sync_copy(m_hbm.at[pl.ds(base, SC_ROWS_PER_W)], m_sl)

    def start(buf, ci):
        pltpu.async_copy(f_hbm.at[pl.ds(base + ci, 1)], f_v.at[pl.ds(buf, 1)],
                         sems[buf])

    def wait(buf):
        pltpu.make_async_copy(f_hbm.at[pl.ds(0, 1)], f_v.at[pl.ds(buf, 1)],
                              sems[buf]).wait()

    def compute(buf, ci):
        civ = jnp.full((16,), ci, jnp.int32)

        @pl.loop(0, D // 4,
                 init_carry=tuple(jnp.zeros((16,), jnp.float32)
                                  for _ in range(NB)))
        def p1(i, cs):
            d0 = i * 4
            dv0 = jnp.full((16,), d0, jnp.int32)
            dvs = [dv0, dv0 + 1, dv0 + 2, dv0 + 3]
            xs = [plsc.load_gather(x_sl, [civ, dv]) for dv in dvs]
            out = []
            for lb in range(NB):
                g = [plsc.load_gather(f_v, [bufv[buf], l_idx[lb], dv])
                     for dv in dvs]
                t = (g[0] * xs[0] + g[1] * xs[1]) + (g[2] * xs[2] + g[3] * xs[3])
                out.append(cs[lb] + t)
            return tuple(out)

        cs = p1

        m_vec = cs[0]
        for lb in range(1, NB):
            m_vec = jnp.maximum(m_vec, cs[lb])
        mx = lax.broadcast_in_dim(jnp.max(m_vec), (16,), ())
        es = [jnp.exp(cs[lb] - mx) for lb in range(NB)]
        s_vec = es[NB - 1] * validf
        for lb in range(NB - 1):
            s_vec = s_vec + es[lb]
        s = lax.broadcast_in_dim(jnp.sum(s_vec), (16,), ())
        sinv = ones / s
        for lb in range(NB):
            mf = m_sl[ci, pl.ds(lb * 16, 16)]
            w_v[pl.ds(lb * 16, 16)] = es[lb] * mf * sinv

        @pl.loop(0, L // 4,
                 init_carry=tuple(jnp.zeros((16,), jnp.float32)
                                  for _ in range(4 * NG)))
        def p2(i, accs):
            l0 = i * 4
            lv0 = jnp.full((16,), l0, jnp.int32)
            accs = list(accs)
            for j in range(4):
                ws = plsc.load_gather(w_v, [lv0 + j]) if j else \
                    plsc.load_gather(w_v, [lv0])
                for g in range(NG):
                    accs[j * NG + g] = accs[j * NG + g] + ws * f_v[
                        buf, l0 + j, pl.ds(g * 16, 16)
                    ]
            return tuple(accs)

        accs = p2
        for g in range(NG):
            o_sl[ci, pl.ds(g * 16, 16)] = (
                accs[g] + accs[NG + g] + accs[2 * NG + g] + accs[3 * NG + g]
            )

    start(0, 0)

    @pl.loop(0, SC_ROWS_PER_W, step=2)
    def _chunk(ci):
        @pl.when(ci + 1 < SC_ROWS_PER_W)
        def _():
            start(1, ci + 1)
        wait(0)
        compute(0, ci)

        @pl.when(ci + 2 < SC_ROWS_PER_W)
        def _():
            start(0, ci + 2)

        @pl.when(ci + 1 < SC_ROWS_PER_W)
        def _():
            wait(1)
            compute(1, ci + 1)

    pltpu.sync_copy(o_sl, out_hbm.at[pl.ds(base, SC_ROWS_PER_W)])


def _sc_attn(f, x, mpad):
    mesh = plsc.VectorSubcoreMesh(
        core_axis_name="c", subcore_axis_name="s", num_cores=2, num_subcores=16
    )
    run = pl.kernel(
        _sc_body,
        out_type=jax.ShapeDtypeStruct((SC_ROWS, D), jnp.float32),
        mesh=mesh,
        scratch_types=[
            pltpu.VMEM((2, L, D), jnp.float32),
            pltpu.VMEM((SC_ROWS_PER_W, D), jnp.float32),
            pltpu.VMEM((SC_ROWS_PER_W, LP), jnp.float32),
            pltpu.VMEM((SC_ROWS_PER_W, D), jnp.float32),
            pltpu.VMEM((LP,), jnp.float32),
            pltpu.SemaphoreType.DMA,
            pltpu.SemaphoreType.DMA,
        ],
        compiler_params=pltpu.CompilerParams(needs_layout_passes=False),
    )
    return run(f, x, mpad)


@jax.jit
def _friendattn(f, x, m):
    mf = m.astype(jnp.float32)
    parts = []
    if S:
        parts.append(_tc_attn(f[:S], x[:S], mf[:S], rows=S))
    if SC_ROWS:
        mpad = jnp.pad(mf[S:], ((0, 0), (0, LP - L)))
        parts.append(_sc_attn(f[S:], x[S:], mpad))
    out = parts[0] if len(parts) == 1 else jnp.concatenate(parts, axis=0)
    return out.reshape(B, 1, D)


def kernel(friend_diff_x, self_x, friend_num_src, friend_num_src_tensor, friend_diff_src_mask):
    del friend_num_src, friend_num_src_tensor
    return _friendattn(friend_diff_x, self_x, friend_diff_src_mask)
